# baseline (speedup 1.0000x reference)
import functools

import jax
import jax.numpy as jnp
from jax import lax
from jax.experimental import pallas as pl
from jax.experimental.pallas import tpu as pltpu
from jax.experimental.pallas import tpu_sc as plsc

N_NODES = 10000
N_SUBCORES = 16
EDGES_PER_TILE = 10112
CHUNK = 128
N_CHUNKS = EDGES_PER_TILE // CHUNK
WIDTH = 64
DEGW = 16
ACC_ROWS = 10112
ZROWS = ACC_ROWS // N_SUBCORES
TROWS = 632


def _make_segsum(tabs_per_core, with_deg, nbuf=2):
    mesh = plsc.VectorSubcoreMesh(core_axis_name="c", subcore_axis_name="s")
    n_tabs = 2 * tabs_per_core
    n_in = n_tabs + 3 + (2 if with_deg else 0)
    n_out = n_tabs + (2 if with_deg else 0)

    scratch = (
        [pltpu.VMEM((N_CHUNKS, CHUNK), jnp.int32),
         pltpu.VMEM((N_CHUNKS, CHUNK), jnp.int32)]
        + [pltpu.VMEM((CHUNK, WIDTH), jnp.float32)] * nbuf
        + [pltpu.VMEM_SHARED((N_NODES, WIDTH), jnp.float32),
           pltpu.VMEM_SHARED((ACC_ROWS, WIDTH), jnp.float32)]
        + [pltpu.SemaphoreType.DMA] * (2 * nbuf)
    )
    if with_deg:
        scratch += [
            pltpu.VMEM((CHUNK, DEGW), jnp.float32),
            pltpu.VMEM_SHARED((ACC_ROWS, DEGW), jnp.float32),
        ]

    @functools.partial(
        pl.kernel,
        out_type=[jax.ShapeDtypeStruct(
            (ACC_ROWS, DEGW if with_deg and i >= n_tabs else WIDTH),
            jnp.float32) for i in range(n_out)],
        mesh=mesh,
        compiler_params=pltpu.CompilerParams(use_tc_tiling_on_sc=False),
        scratch_types=scratch,
    )
    def segsum(*args):
        tabs = args[:n_tabs]
        srcs, dsts, zeros = args[n_tabs:n_tabs + 3]
        if with_deg:
            ones, zeros_d = args[n_tabs + 3:n_in]
        outs = args[n_in:n_in + n_tabs]
        rest = args[n_in + n_tabs:]
        if with_deg:
            deg_outs = rest[:2]
            rest = rest[2:]
        src_v, dst_v = rest[:2]
        gb = rest[2:2 + nbuf]
        tab_s, acc = rest[2 + nbuf:4 + nbuf]
        gsem = rest[4 + nbuf:4 + 2 * nbuf]
        ssem = rest[4 + 2 * nbuf:4 + 3 * nbuf]
        if with_deg:
            ones_v, acc_d = rest[4 + 3 * nbuf:]
        c = lax.axis_index("c")
        s = lax.axis_index("s")
        pltpu.sync_copy(srcs.at[s], src_v)
        pltpu.sync_copy(dsts.at[s], dst_v)

        def stage(tab):
            @pl.when(s < N_SUBCORES - 1)
            def _():
                pltpu.sync_copy(tab.at[pl.ds(s * TROWS, TROWS)],
                                tab_s.at[pl.ds(s * TROWS, TROWS)])

            @pl.when(s == N_SUBCORES - 1)
            def _():
                pltpu.sync_copy(
                    tab.at[pl.ds((N_SUBCORES - 1) * TROWS,
                                 N_NODES - (N_SUBCORES - 1) * TROWS)],
                    tab_s.at[pl.ds((N_SUBCORES - 1) * TROWS,
                                   N_NODES - (N_SUBCORES - 1) * TROWS)])

        def gather(j, b):
            pltpu.async_copy(tab_s.at[src_v.at[j]], gb[b], gsem[b])

        def gwait(b):
            pltpu.make_async_copy(tab_s.at[src_v.at[0]], gb[b],
                                  gsem[b]).wait()

        def swait(b):
            pltpu.make_async_copy(gb[b], acc.at[dst_v.at[0]],
                                  ssem[b]).wait()

        def one_pass(tab, out):
            stage(tab)
            pltpu.sync_copy(zeros, acc.at[pl.ds(s * ZROWS, ZROWS)])
            plsc.subcore_barrier()

            if nbuf == 2:
                gather(0, 0)

                def body(i, carry):
                    j = 2 * i
                    gather(j + 1, 1)
                    gwait(0)
                    pltpu.sync_copy(gb[0], acc.at[dst_v.at[j]], add=True)
                    gather(j + 2, 0)
                    gwait(1)
                    pltpu.sync_copy(gb[1], acc.at[dst_v.at[j + 1]], add=True)
                    return carry

                lax.fori_loop(0, N_CHUNKS // 2, body, 0)
                gwait(0)
                pltpu.sync_copy(gb[0], acc.at[dst_v.at[N_CHUNKS - 1]],
                                add=True)
            else:
                gather(0, 0)
                gather(1, 1)

                def body(i, carry):
                    for k in range(3):
                        j = 3 * i + k
                        gwait(k)
                        pltpu.async_copy(gb[k], acc.at[dst_v.at[j]],
                                         ssem[k], add=True)

                        @pl.when(j >= 1)
                        def _(k=k):
                            swait((k + 2) % 3)

                        @pl.when(j + 2 < N_CHUNKS)
                        def _(j=j, k=k):
                            gather(j + 2, (k + 2) % 3)
                    return carry

                lax.fori_loop(0, N_CHUNKS // 3, body, 0)
                gwait(0)
                pltpu.async_copy(gb[0], acc.at[dst_v.at[N_CHUNKS - 1]],
                                 ssem[0], add=True)
                swait(2)
                swait(0)
            plsc.subcore_barrier()
            pltpu.sync_copy(acc.at[pl.ds(s * ZROWS, ZROWS)],
                            out.at[pl.ds(s * ZROWS, ZROWS)])

        def deg_pass(lo, hi, out_d):
            pltpu.sync_copy(ones, ones_v)
            pltpu.sync_copy(zeros_d, acc_d.at[pl.ds(s * ZROWS, ZROWS)])
            plsc.subcore_barrier()

            def body(j, carry):
                pltpu.sync_copy(ones_v, acc_d.at[dst_v.at[j]], add=True)
                return carry

            lax.fori_loop(lo, hi, body, 0)
            plsc.subcore_barrier()
            pltpu.sync_copy(acc_d.at[pl.ds(s * ZROWS, ZROWS)],
                            out_d.at[pl.ds(s * ZROWS, ZROWS)])

        for t in range(tabs_per_core):
            @pl.when(c == 0)
            def _(t=t):
                one_pass(tabs[t], outs[t])

            @pl.when(c == 1)
            def _(t=t):
                one_pass(tabs[tabs_per_core + t], outs[tabs_per_core + t])
            plsc.subcore_barrier()

        if with_deg:
            half = N_CHUNKS // 2

            @pl.when(c == 0)
            def _():
                deg_pass(0, half, deg_outs[0])

            @pl.when(c == 1)
            def _():
                deg_pass(half, N_CHUNKS, deg_outs[1])

    return segsum


def _dense_body(n_agg, n_xr, n_out, relu, *refs):
    aggs = refs[:n_agg]
    d0_ref, d1_ref = refs[n_agg:n_agg + 2]
    xrs = refs[n_agg + 2:n_agg + 2 + n_xr]
    wl_ref, wr_ref, b_ref = refs[n_agg + 2 + n_xr:n_agg + 5 + n_xr]
    outs = refs[n_agg + 5 + n_xr:]
    deg = d0_ref[:, 0:1] + d1_ref[:, 0:1]
    inv = 1.0 / jnp.maximum(deg, 1.0)
    mean = jnp.concatenate([a[...] * inv for a in aggs], axis=1)
    xcat = (xrs[0][...] if n_xr == 1 else
            jnp.concatenate([r[...] for r in xrs], axis=1))
    acc = jnp.dot(mean.astype(jnp.bfloat16), wl_ref[...],
                  preferred_element_type=jnp.float32)
    acc = acc + jnp.dot(xcat.astype(jnp.bfloat16), wr_ref[...],
                        preferred_element_type=jnp.float32)
    acc = acc + b_ref[...]
    if relu:
        acc = jnp.maximum(acc, 0.0)
    if n_out == 1:
        outs[0][...] = acc
    else:
        w = acc.shape[1] // n_out
        for k in range(n_out):
            outs[k][...] = acc[:, k * w:(k + 1) * w]


def _dense_layer(aggs, d0, d1, xrs, wl, wr, bias, relu, n_out, mb=2000):
    wl = wl.astype(jnp.bfloat16)
    wr = wr.astype(jnp.bfloat16)
    m = xrs[0].shape[0]
    k = wl.shape[0]
    n = wl.shape[1]
    wa = aggs[0].shape[1]
    wx = xrs[0].shape[1]
    out_shape = [jax.ShapeDtypeStruct((m, n // n_out), jnp.float32)
                 for _ in range(n_out)]
    res = pl.pallas_call(
        functools.partial(_dense_body, len(aggs), len(xrs), n_out, relu),
        grid=(m // mb,),
        in_specs=(
            [pl.BlockSpec((mb, wa), lambda i: (i, 0)) for _ in aggs]
            + [pl.BlockSpec((mb, DEGW), lambda i: (i, 0))] * 2
            + [pl.BlockSpec((mb, wx), lambda i: (i, 0)) for _ in xrs]
            + [pl.BlockSpec((k, n), lambda i: (0, 0)),
               pl.BlockSpec((k, n), lambda i: (0, 0)),
               pl.BlockSpec((1, n), lambda i: (0, 0))]
        ),
        out_specs=[pl.BlockSpec((mb, n // n_out), lambda i: (i, 0))
                   for _ in range(n_out)],
        out_shape=out_shape,
    )(*aggs, d0, d1, *xrs, wl, wr, bias)
    return res


def kernel(x, edge_index, W1_l, b1, W1_r, W2_l, b2, W2_r):
    src = edge_index[0].astype(jnp.int32)
    dst = edge_index[1].astype(jnp.int32)
    n_edges = src.shape[0]

    e_pad = N_SUBCORES * EDGES_PER_TILE - n_edges
    src_p = jnp.concatenate([src, jnp.zeros((e_pad,), jnp.int32)])
    dst_p = jnp.concatenate([dst, jnp.full((e_pad,), N_NODES, jnp.int32)])
    srcs = src_p.reshape(N_SUBCORES, N_CHUNKS, CHUNK)
    dsts = dst_p.reshape(N_SUBCORES, N_CHUNKS, CHUNK)

    zeros = jnp.zeros((ZROWS, WIDTH), jnp.float32)
    zeros_d = jnp.zeros((ZROWS, DEGW), jnp.float32)
    ones = jnp.ones((CHUNK, DEGW), jnp.float32)

    tabs1 = [x[:, k * WIDTH:(k + 1) * WIDTH] for k in range(4)]
    o0, o1, o2, o3, d0, d1 = _make_segsum(2, True)(
        *tabs1, srcs, dsts, zeros, ones, zeros_d)
    hs = _dense_layer([o0, o1, o2, o3], d0, d1, [x], W1_l, W1_r,
                      b1.reshape(1, -1), relu=True, n_out=8)

    a2 = _make_segsum(4, False, nbuf=3)(*hs, srcs, dsts, zeros)
    out, = _dense_layer(list(a2), d0, d1, hs, W2_l, W2_r, b2.reshape(1, -1),
                        relu=False, n_out=1)
    return out


# speedup vs baseline: 1.0006x; 1.0006x over previous
import functools

import jax
import jax.numpy as jnp
from jax import lax
from jax.experimental import pallas as pl
from jax.experimental.pallas import tpu as pltpu
from jax.experimental.pallas import tpu_sc as plsc

N_NODES = 10000
N_SUBCORES = 16
EDGES_PER_TILE = 10112
CHUNK = 128
N_CHUNKS = EDGES_PER_TILE // CHUNK
WIDTH = 64
DEGW = 16
ACC_ROWS = 10112
ZROWS = ACC_ROWS // N_SUBCORES
TROWS = 632


def _make_segsum(tabs_per_core, with_deg, nbuf=2):
    mesh = plsc.VectorSubcoreMesh(core_axis_name="c", subcore_axis_name="s")
    n_tabs = 2 * tabs_per_core
    n_in = n_tabs + 3 + (2 if with_deg else 0)
    n_out = n_tabs + (2 if with_deg else 0)

    scratch = (
        [pltpu.VMEM((N_CHUNKS, CHUNK), jnp.int32),
         pltpu.VMEM((N_CHUNKS, CHUNK), jnp.int32)]
        + [pltpu.VMEM((CHUNK, WIDTH), jnp.float32)] * nbuf
        + [pltpu.VMEM_SHARED((N_NODES, WIDTH), jnp.float32),
           pltpu.VMEM_SHARED((ACC_ROWS, WIDTH), jnp.float32)]
        + [pltpu.SemaphoreType.DMA] * (2 * nbuf)
    )
    if with_deg:
        scratch += [
            pltpu.VMEM((CHUNK, DEGW), jnp.float32),
            pltpu.VMEM_SHARED((ACC_ROWS, DEGW), jnp.float32),
        ]

    @functools.partial(
        pl.kernel,
        out_type=[jax.ShapeDtypeStruct(
            (ACC_ROWS, DEGW if with_deg and i >= n_tabs else WIDTH),
            jnp.float32) for i in range(n_out)],
        mesh=mesh,
        compiler_params=pltpu.CompilerParams(use_tc_tiling_on_sc=False),
        scratch_types=scratch,
    )
    def segsum(*args):
        tabs = args[:n_tabs]
        srcs, dsts, zeros = args[n_tabs:n_tabs + 3]
        if with_deg:
            ones, zeros_d = args[n_tabs + 3:n_in]
        outs = args[n_in:n_in + n_tabs]
        rest = args[n_in + n_tabs:]
        if with_deg:
            deg_outs = rest[:2]
            rest = rest[2:]
        src_v, dst_v = rest[:2]
        gb = rest[2:2 + nbuf]
        tab_s, acc = rest[2 + nbuf:4 + nbuf]
        gsem = rest[4 + nbuf:4 + 2 * nbuf]
        ssem = rest[4 + 2 * nbuf:4 + 3 * nbuf]
        if with_deg:
            ones_v, acc_d = rest[4 + 3 * nbuf:]
        c = lax.axis_index("c")
        s = lax.axis_index("s")
        pltpu.sync_copy(srcs.at[s], src_v)
        pltpu.sync_copy(dsts.at[s], dst_v)

        def stage(tab):
            @pl.when(s < N_SUBCORES - 1)
            def _():
                pltpu.sync_copy(tab.at[pl.ds(s * TROWS, TROWS)],
                                tab_s.at[pl.ds(s * TROWS, TROWS)])

            @pl.when(s == N_SUBCORES - 1)
            def _():
                pltpu.sync_copy(
                    tab.at[pl.ds((N_SUBCORES - 1) * TROWS,
                                 N_NODES - (N_SUBCORES - 1) * TROWS)],
                    tab_s.at[pl.ds((N_SUBCORES - 1) * TROWS,
                                   N_NODES - (N_SUBCORES - 1) * TROWS)])

        def gather(j, b):
            pltpu.async_copy(tab_s.at[src_v.at[j]], gb[b], gsem[b])

        def gwait(b):
            pltpu.make_async_copy(tab_s.at[src_v.at[0]], gb[b],
                                  gsem[b]).wait()

        def swait(b):
            pltpu.make_async_copy(gb[b], acc.at[dst_v.at[0]],
                                  ssem[b]).wait()

        def one_pass(tab, out):
            stage(tab)
            pltpu.sync_copy(zeros, acc.at[pl.ds(s * ZROWS, ZROWS)])
            plsc.subcore_barrier()

            if nbuf == 2:
                gather(0, 0)

                def body(i, carry):
                    j = 2 * i
                    gather(j + 1, 1)
                    gwait(0)
                    pltpu.sync_copy(gb[0], acc.at[dst_v.at[j]], add=True)
                    gather(j + 2, 0)
                    gwait(1)
                    pltpu.sync_copy(gb[1], acc.at[dst_v.at[j + 1]], add=True)
                    return carry

                lax.fori_loop(0, N_CHUNKS // 2, body, 0)
                gwait(0)
                pltpu.sync_copy(gb[0], acc.at[dst_v.at[N_CHUNKS - 1]],
                                add=True)
            else:
                gather(0, 0)
                gather(1, 1)

                def body(i, carry):
                    for k in range(3):
                        j = 3 * i + k
                        gwait(k)
                        pltpu.async_copy(gb[k], acc.at[dst_v.at[j]],
                                         ssem[k], add=True)

                        @pl.when(j >= 1)
                        def _(k=k):
                            swait((k + 2) % 3)

                        @pl.when(j + 2 < N_CHUNKS)
                        def _(j=j, k=k):
                            gather(j + 2, (k + 2) % 3)
                    return carry

                lax.fori_loop(0, N_CHUNKS // 3, body, 0)
                gwait(0)
                pltpu.async_copy(gb[0], acc.at[dst_v.at[N_CHUNKS - 1]],
                                 ssem[0], add=True)
                swait(2)
                swait(0)
            plsc.subcore_barrier()
            pltpu.sync_copy(acc.at[pl.ds(s * ZROWS, ZROWS)],
                            out.at[pl.ds(s * ZROWS, ZROWS)])

        def deg_pass(lo, hi, out_d):
            pltpu.sync_copy(ones, ones_v)
            pltpu.sync_copy(zeros_d, acc_d.at[pl.ds(s * ZROWS, ZROWS)])
            plsc.subcore_barrier()

            def body(j, carry):
                pltpu.sync_copy(ones_v, acc_d.at[dst_v.at[j]], add=True)
                return carry

            lax.fori_loop(lo, hi, body, 0)
            plsc.subcore_barrier()
            pltpu.sync_copy(acc_d.at[pl.ds(s * ZROWS, ZROWS)],
                            out_d.at[pl.ds(s * ZROWS, ZROWS)])

        for t in range(tabs_per_core):
            @pl.when(c == 0)
            def _(t=t):
                one_pass(tabs[t], outs[t])

            @pl.when(c == 1)
            def _(t=t):
                one_pass(tabs[tabs_per_core + t], outs[tabs_per_core + t])
            plsc.subcore_barrier()

        if with_deg:
            half = N_CHUNKS // 2

            @pl.when(c == 0)
            def _():
                deg_pass(0, half, deg_outs[0])

            @pl.when(c == 1)
            def _():
                deg_pass(half, N_CHUNKS, deg_outs[1])

    return segsum


def _dense_body(n_agg, n_xr, n_out, relu, *refs):
    aggs = refs[:n_agg]
    d0_ref, d1_ref = refs[n_agg:n_agg + 2]
    xrs = refs[n_agg + 2:n_agg + 2 + n_xr]
    wl_ref, wr_ref, b_ref = refs[n_agg + 2 + n_xr:n_agg + 5 + n_xr]
    outs = refs[n_agg + 5 + n_xr:]
    deg = d0_ref[:, 0:1] + d1_ref[:, 0:1]
    inv = 1.0 / jnp.maximum(deg, 1.0)
    mean = jnp.concatenate([a[...] * inv for a in aggs], axis=1)
    xcat = (xrs[0][...] if n_xr == 1 else
            jnp.concatenate([r[...] for r in xrs], axis=1))
    acc = jnp.dot(mean, wl_ref[...], preferred_element_type=jnp.float32)
    acc = acc + jnp.dot(xcat, wr_ref[...],
                        preferred_element_type=jnp.float32)
    acc = acc + b_ref[...]
    if relu:
        acc = jnp.maximum(acc, 0.0)
    if n_out == 1:
        outs[0][...] = acc
    else:
        w = acc.shape[1] // n_out
        for k in range(n_out):
            outs[k][...] = acc[:, k * w:(k + 1) * w]


def _dense_layer(aggs, d0, d1, xrs, wl, wr, bias, relu, n_out, mb=1000):
    m = xrs[0].shape[0]
    k = wl.shape[0]
    n = wl.shape[1]
    wa = aggs[0].shape[1]
    wx = xrs[0].shape[1]
    out_shape = [jax.ShapeDtypeStruct((m, n // n_out), jnp.float32)
                 for _ in range(n_out)]
    res = pl.pallas_call(
        functools.partial(_dense_body, len(aggs), len(xrs), n_out, relu),
        grid=(m // mb,),
        in_specs=(
            [pl.BlockSpec((mb, wa), lambda i: (i, 0)) for _ in aggs]
            + [pl.BlockSpec((mb, DEGW), lambda i: (i, 0))] * 2
            + [pl.BlockSpec((mb, wx), lambda i: (i, 0)) for _ in xrs]
            + [pl.BlockSpec((k, n), lambda i: (0, 0)),
               pl.BlockSpec((k, n), lambda i: (0, 0)),
               pl.BlockSpec((1, n), lambda i: (0, 0))]
        ),
        out_specs=[pl.BlockSpec((mb, n // n_out), lambda i: (i, 0))
                   for _ in range(n_out)],
        out_shape=out_shape,
    )(*aggs, d0, d1, *xrs, wl, wr, bias)
    return res


def kernel(x, edge_index, W1_l, b1, W1_r, W2_l, b2, W2_r):
    src = edge_index[0].astype(jnp.int32)
    dst = edge_index[1].astype(jnp.int32)
    n_edges = src.shape[0]

    e_pad = N_SUBCORES * EDGES_PER_TILE - n_edges
    src_p = jnp.concatenate([src, jnp.zeros((e_pad,), jnp.int32)])
    dst_p = jnp.concatenate([dst, jnp.full((e_pad,), N_NODES, jnp.int32)])
    srcs = src_p.reshape(N_SUBCORES, N_CHUNKS, CHUNK)
    dsts = dst_p.reshape(N_SUBCORES, N_CHUNKS, CHUNK)

    zeros = jnp.zeros((ZROWS, WIDTH), jnp.float32)
    zeros_d = jnp.zeros((ZROWS, DEGW), jnp.float32)
    ones = jnp.ones((CHUNK, DEGW), jnp.float32)

    tabs1 = [x[:, k * WIDTH:(k + 1) * WIDTH] for k in range(4)]
    o0, o1, o2, o3, d0, d1 = _make_segsum(2, True)(
        *tabs1, srcs, dsts, zeros, ones, zeros_d)
    hs = _dense_layer([o0, o1, o2, o3], d0, d1, [x], W1_l, W1_r,
                      b1.reshape(1, -1), relu=True, n_out=8)

    a2 = _make_segsum(4, False, nbuf=3)(*hs, srcs, dsts, zeros)
    out, = _dense_layer(list(a2), d0, d1, hs, W2_l, W2_r, b2.reshape(1, -1),
                        relu=False, n_out=1)
    return out
